# Initial kernel scaffold; baseline (speedup 1.0000x reference)
import jax
import jax.numpy as jnp
from jax.experimental import pallas as pl


def kernel(zeros_loc, ones_loc, zeros_idx):
    raise NotImplementedError("write your pallas kernel here")


# speedup vs baseline: 1.1829x; 1.1829x over previous
import functools

import jax
import jax.numpy as jnp
from jax import lax
from jax.experimental import pallas as pl
from jax.experimental.pallas import tpu as pltpu
from jax.experimental.pallas import tpu_sc as plsc

H = 512
W = 512
Z = 16384
O = 4096

QBLK = 512


def _dist_body(z_ref, o_ref, out_ref):
    o = o_ref[...]
    z = z_ref[...]
    a = lax.dot_general(o, z, (((1,), (0,)), ((), ())),
                        preferred_element_type=jnp.float32)
    b = jnp.sum(o * o, axis=1, keepdims=True)
    t = b - (a + a)
    m = jnp.min(t, axis=0, keepdims=True)
    c = z[0:1, :] * z[0:1, :] + z[1:2, :] * z[1:2, :]
    out_ref[...] = jnp.sqrt(jnp.maximum(m + c, 1e-6))


def _min_dists(zeros_loc, ones_loc):
    zt = zeros_loc.T
    return pl.pallas_call(
        _dist_body,
        grid=(Z // QBLK,),
        in_specs=[
            pl.BlockSpec((2, QBLK), lambda i: (0, i)),
            pl.BlockSpec((O, 2), lambda i: (0, 0)),
        ],
        out_specs=pl.BlockSpec((1, QBLK), lambda i: (0, i)),
        out_shape=jax.ShapeDtypeStruct((1, Z), jnp.float32),
    )(zt, ones_loc)


_NTILES = 16
_ROWS = Z // 128
_RPW = _ROWS // _NTILES
_ZSLICE = H * W // _NTILES
_ZBUF = 2048


def _scatter_body(d_hbm, r_hbm, c_hbm, out_hbm, r_v, c_v, idx_v, val_v,
                  zero_v, sem):
    sid = lax.axis_index("s")
    zv = jnp.zeros((16,), jnp.float32)
    for i in range(_ZBUF // 16):
        zero_v[pl.ds(i * 16, 16)] = zv
    zbase = sid * _ZSLICE
    for j in range(_ZSLICE // _ZBUF):
        pltpu.sync_copy(zero_v, out_hbm.at[pl.ds(zbase + j * _ZBUF, _ZBUF)])
    row0 = sid * _RPW
    pltpu.sync_copy(r_hbm.at[pl.ds(row0, _RPW)], r_v)
    pltpu.sync_copy(c_hbm.at[pl.ds(row0, _RPW)], c_v)
    pltpu.sync_copy(d_hbm.at[pl.ds(row0, _RPW)], val_v)
    for j in range(_RPW):
        for i in range(128 // 16):
            s = pl.ds(i * 16, 16)
            idx_v[j, s] = r_v[j, s] * W + c_v[j, s]
    plsc.subcore_barrier()
    for j in range(_RPW):
        pltpu.async_copy(val_v.at[j], out_hbm.at[idx_v.at[j]], sem).wait()


def _scatter(dists2, rows2, cols2):
    mesh = plsc.VectorSubcoreMesh(core_axis_name="c", subcore_axis_name="s",
                                  num_cores=1)
    run = functools.partial(
        pl.kernel,
        out_type=jax.ShapeDtypeStruct((H * W,), jnp.float32),
        mesh=mesh,
        scratch_types=[
            pltpu.VMEM((_RPW, 128), jnp.int32),
            pltpu.VMEM((_RPW, 128), jnp.int32),
            pltpu.VMEM((_RPW, 128), jnp.int32),
            pltpu.VMEM((_RPW, 128), jnp.float32),
            pltpu.VMEM((_ZBUF,), jnp.float32),
            pltpu.SemaphoreType.DMA,
        ],
    )(_scatter_body)
    return run(dists2, rows2, cols2)


def kernel(zeros_loc, ones_loc, zeros_idx):
    dists = _min_dists(zeros_loc, ones_loc)
    d2 = dists.reshape(_ROWS, 128)
    r2 = zeros_idx[:, 0].reshape(_ROWS, 128)
    c2 = zeros_idx[:, 1].reshape(_ROWS, 128)
    return _scatter(d2, r2, c2).reshape(H, W)


# speedup vs baseline: 1.2114x; 1.0241x over previous
import functools

import jax
import jax.numpy as jnp
from jax import lax
from jax.experimental import pallas as pl
from jax.experimental.pallas import tpu as pltpu
from jax.experimental.pallas import tpu_sc as plsc

H = 512
W = 512
Z = 16384
O = 4096

QBLK = 512


def _dist_body(z_ref, o_ref, out_ref):
    o2 = o_ref[...]
    z = z_ref[...]
    a = lax.dot_general(o2, z, (((1,), (0,)), ((), ())),
                        preferred_element_type=jnp.float32)
    b = jnp.sum(o2 * o2, axis=1, keepdims=True) * 0.25
    m = jnp.min(a + b, axis=0, keepdims=True)
    c = z[0:1, :] * z[0:1, :] + z[1:2, :] * z[1:2, :]
    out_ref[...] = jnp.sqrt(jnp.maximum(m + c, 1e-6))


def _min_dists(zeros_loc, ones_loc):
    zt = zeros_loc.T
    return pl.pallas_call(
        _dist_body,
        grid=(Z // QBLK,),
        in_specs=[
            pl.BlockSpec((2, QBLK), lambda i: (0, i)),
            pl.BlockSpec((O, 2), lambda i: (0, 0)),
        ],
        out_specs=pl.BlockSpec((1, QBLK), lambda i: (0, i)),
        out_shape=jax.ShapeDtypeStruct((1, Z), jnp.float32),
    )(zt, ones_loc * -2.0)


_NTILES = 16
_ROWS = Z // 128
_RPW = _ROWS // _NTILES
_ZSLICE = H * W // _NTILES
_ZBUF = 4096


def _scatter_body(d_hbm, r_hbm, c_hbm, out_hbm, r_v, c_v, idx_v, val_v,
                  zero_v, sem, zsem):
    sid = lax.axis_index("s")
    row0 = sid * _RPW
    in_cps = [
        pltpu.async_copy(r_hbm.at[pl.ds(row0, _RPW)], r_v, sem),
        pltpu.async_copy(c_hbm.at[pl.ds(row0, _RPW)], c_v, sem),
        pltpu.async_copy(d_hbm.at[pl.ds(row0, _RPW)], val_v, sem),
    ]
    zv = jnp.zeros((16,), jnp.float32)
    for i in range(_ZBUF // 16):
        zero_v[pl.ds(i * 16, 16)] = zv
    zbase = sid * _ZSLICE
    zero_cps = [
        pltpu.async_copy(zero_v, out_hbm.at[pl.ds(zbase + j * _ZBUF, _ZBUF)],
                         zsem)
        for j in range(_ZSLICE // _ZBUF)
    ]
    for cp in in_cps:
        cp.wait()
    for j in range(_RPW):
        for i in range(128 // 16):
            s = pl.ds(i * 16, 16)
            idx_v[j, s] = r_v[j, s] * W + c_v[j, s]
    for cp in zero_cps:
        cp.wait()
    plsc.subcore_barrier()
    sc_cps = [
        pltpu.async_copy(val_v.at[j], out_hbm.at[idx_v.at[j]], sem)
        for j in range(_RPW)
    ]
    for cp in sc_cps:
        cp.wait()


def _scatter(dists2, rows2, cols2):
    mesh = plsc.VectorSubcoreMesh(core_axis_name="c", subcore_axis_name="s",
                                  num_cores=1)
    run = functools.partial(
        pl.kernel,
        out_type=jax.ShapeDtypeStruct((H * W,), jnp.float32),
        mesh=mesh,
        scratch_types=[
            pltpu.VMEM((_RPW, 128), jnp.int32),
            pltpu.VMEM((_RPW, 128), jnp.int32),
            pltpu.VMEM((_RPW, 128), jnp.int32),
            pltpu.VMEM((_RPW, 128), jnp.float32),
            pltpu.VMEM((_ZBUF,), jnp.float32),
            pltpu.SemaphoreType.DMA,
            pltpu.SemaphoreType.DMA,
        ],
    )(_scatter_body)
    return run(dists2, rows2, cols2)


def kernel(zeros_loc, ones_loc, zeros_idx):
    dists = _min_dists(zeros_loc, ones_loc)
    d2 = dists.reshape(_ROWS, 128)
    r2 = zeros_idx[:, 0].reshape(_ROWS, 128)
    c2 = zeros_idx[:, 1].reshape(_ROWS, 128)
    return _scatter(d2, r2, c2).reshape(H, W)


# speedup vs baseline: 1.2564x; 1.0371x over previous
import functools

import jax
import jax.numpy as jnp
from jax import lax
from jax.experimental import pallas as pl
from jax.experimental.pallas import tpu as pltpu
from jax.experimental.pallas import tpu_sc as plsc

H = 512
W = 512
Z = 16384
O = 4096

QBLK = 1024


def _dist_body(z_ref, o_ref, out_ref):
    o2 = o_ref[...]
    z = z_ref[...]
    a = lax.dot_general(o2, z, (((1,), (0,)), ((), ())),
                        preferred_element_type=jnp.float32)
    b = jnp.sum(o2 * o2, axis=1, keepdims=True) * 0.25
    m = jnp.min(a + b, axis=0, keepdims=True)
    c = z[0:1, :] * z[0:1, :] + z[1:2, :] * z[1:2, :]
    out_ref[...] = jnp.sqrt(jnp.maximum(m + c, 1e-6))


def _min_dists(zeros_loc, ones_loc):
    zt = zeros_loc.T
    return pl.pallas_call(
        _dist_body,
        grid=(Z // QBLK,),
        in_specs=[
            pl.BlockSpec((2, QBLK), lambda i: (0, i)),
            pl.BlockSpec((O, 2), lambda i: (0, 0)),
        ],
        out_specs=pl.BlockSpec((1, QBLK), lambda i: (0, i)),
        out_shape=jax.ShapeDtypeStruct((1, Z), jnp.float32),
    )(zt, ones_loc * -2.0)


_NTILES = 16
_ROWS = Z // 128
_RPW = _ROWS // _NTILES
_ZSLICE = H * W // _NTILES
_ZBUF = 4096


def _scatter_body(d_hbm, r_hbm, c_hbm, out_hbm, r_v, c_v, idx_v, val_v,
                  zero_v, sem, zsem):
    sid = lax.axis_index("s")
    row0 = sid * _RPW
    in_cps = [
        pltpu.async_copy(r_hbm.at[pl.ds(row0, _RPW)], r_v, sem),
        pltpu.async_copy(c_hbm.at[pl.ds(row0, _RPW)], c_v, sem),
        pltpu.async_copy(d_hbm.at[pl.ds(row0, _RPW)], val_v, sem),
    ]
    zv = jnp.zeros((16,), jnp.float32)
    for i in range(_ZBUF // 16):
        zero_v[pl.ds(i * 16, 16)] = zv
    zbase = sid * _ZSLICE
    zero_cps = [
        pltpu.async_copy(zero_v, out_hbm.at[pl.ds(zbase + j * _ZBUF, _ZBUF)],
                         zsem)
        for j in range(_ZSLICE // _ZBUF)
    ]
    for cp in in_cps:
        cp.wait()
    for j in range(_RPW):
        for i in range(128 // 16):
            s = pl.ds(i * 16, 16)
            idx_v[j, s] = r_v[j, s] * W + c_v[j, s]
    for cp in zero_cps:
        cp.wait()
    plsc.subcore_barrier()
    sc_cps = [
        pltpu.async_copy(val_v.at[j], out_hbm.at[idx_v.at[j]], sem)
        for j in range(_RPW)
    ]
    for cp in sc_cps:
        cp.wait()


def _scatter(dists2, rows2, cols2):
    mesh = plsc.VectorSubcoreMesh(core_axis_name="c", subcore_axis_name="s",
                                  num_cores=1)
    run = functools.partial(
        pl.kernel,
        out_type=jax.ShapeDtypeStruct((H * W,), jnp.float32),
        mesh=mesh,
        scratch_types=[
            pltpu.VMEM((_RPW, 128), jnp.int32),
            pltpu.VMEM((_RPW, 128), jnp.int32),
            pltpu.VMEM((_RPW, 128), jnp.int32),
            pltpu.VMEM((_RPW, 128), jnp.float32),
            pltpu.VMEM((_ZBUF,), jnp.float32),
            pltpu.SemaphoreType.DMA,
            pltpu.SemaphoreType.DMA,
        ],
    )(_scatter_body)
    return run(dists2, rows2, cols2)


def kernel(zeros_loc, ones_loc, zeros_idx):
    dists = _min_dists(zeros_loc, ones_loc)
    d2 = dists.reshape(_ROWS, 128)
    r2 = zeros_idx[:, 0].reshape(_ROWS, 128)
    c2 = zeros_idx[:, 1].reshape(_ROWS, 128)
    return _scatter(d2, r2, c2).reshape(H, W)
